# baseline (speedup 1.0000x reference)
import functools

import jax
import jax.numpy as jnp
from jax import lax
from jax.experimental import pallas as pl
from jax.experimental.pallas import tpu as pltpu
from jax.experimental.pallas import tpu_sc as plsc

_ATOMIC_NUMBERS = [0.0, 1.0, 6.0, 7.0, 8.0, 9.0, 15.0, 16.0, 17.0, 35.0, 53.0, 5.0, 14.0, 34.0, 33.0, 26.0, 30.0, 20.0, 12.0, 11.0, 19.0, 25.0, 29.0]
_ELECTRONEG = [2.0, 2.2, 2.55, 3.04, 3.44, 3.98, 2.19, 2.58, 3.16, 2.96, 2.66, 2.04, 1.9, 2.55, 2.18, 1.83, 1.65, 1.0, 1.31, 0.93, 0.82, 1.55, 1.9]
_RADII = [1.7, 1.2, 1.7, 1.55, 1.52, 1.47, 1.8, 1.8, 1.75, 1.85, 1.98, 1.92, 2.1, 1.9, 1.85, 2.0, 1.39, 2.31, 1.73, 2.27, 2.75, 2.05, 1.4]

_D = 128
_D4 = 32
_NT, _NH, _NC = 23, 5, 11
_TROWS = _NT * _NH * _NC
_TPAD = 1280
_NCORE, _NSUB, _LANES = 2, 16, 16
_NW = _NCORE * _NSUB
_CH = 160


def _table_body(aw_ref, hw_ref, cw_ref, props_ref, pw_ref, pb_ref, ow_ref,
                ob_ref, table_ref):
    ow = ow_ref[...]
    w1 = ow[:, 0 * _D4:1 * _D4]
    w2 = ow[:, 1 * _D4:2 * _D4]
    w3 = ow[:, 2 * _D4:3 * _D4]
    w4 = ow[:, 3 * _D4:4 * _D4]
    f32 = jnp.float32
    dot = functools.partial(jnp.dot, preferred_element_type=f32)
    prop_emb = dot(props_ref[...], pw_ref[...].T) + pb_ref[...].reshape(1, _D4)
    a_tab = (dot(aw_ref[...], w1.T) + dot(prop_emb, w2.T)
             + ob_ref[...].reshape(1, _D))
    h_tab = dot(hw_ref[...], w3.T)
    c_tab = dot(cw_ref[...], w4.T)
    a_x = jnp.broadcast_to(a_tab[:, None, :], (_NT, _NH * _NC, _D))
    h_x = jnp.broadcast_to(h_tab[:, None, :], (_NH, _NC, _D)).reshape(
        _NH * _NC, _D)
    c_x = (h_x + jnp.broadcast_to(c_tab[None, :, :], (_NH, _NC, _D)).reshape(
        _NH * _NC, _D))
    full = (a_x + c_x[None, :, :]).reshape(_TROWS, _D)
    table_ref[...] = jnp.concatenate(
        [full, jnp.zeros((_TPAD - _TROWS, _D), f32)], axis=0)


def _make_sc_gather(npad, nout):
    apw = npad // _NW
    nch = apw // _CH
    w0 = nout // apw
    rem = nout % apw
    mesh = plsc.VectorSubcoreMesh(core_axis_name="c", subcore_axis_name="s")

    @functools.partial(
        pl.kernel,
        mesh=mesh,
        out_type=jax.ShapeDtypeStruct((nout, _D), jnp.float32),
        scratch_types=[
            pltpu.VMEM((apw,), jnp.int32),
            pltpu.VMEM((apw,), jnp.int32),
            pltpu.VMEM((apw,), jnp.int32),
            pltpu.VMEM((apw,), jnp.int32),
            pltpu.VMEM((_CH, _D), jnp.float32),
            pltpu.VMEM((_CH, _D), jnp.float32),
            pltpu.VMEM((_CH, _D), jnp.float32),
            pltpu.VMEM((_CH, _D), jnp.float32),
            pltpu.VMEM_SHARED((_TPAD, _D), jnp.float32),
            pltpu.SemaphoreType.DMA,
            pltpu.SemaphoreType.DMA,
            pltpu.SemaphoreType.DMA,
            pltpu.SemaphoreType.DMA,
            pltpu.SemaphoreType.DMA,
            pltpu.SemaphoreType.DMA,
            pltpu.SemaphoreType.DMA,
            pltpu.SemaphoreType.DMA,
        ],
    )
    def sc_gather(table_hbm, at_hbm, hy_hbm, fc_hbm, out_hbm,
                  atv, hyv, fcv, idx_v, buf0, buf1, buf2, buf3, tab_sh,
                  gsem0, gsem1, gsem2, gsem3, ssem0, ssem1, ssem2, ssem3):
        sid = lax.axis_index("s")
        wid = sid * _NCORE + lax.axis_index("c")
        base = wid * apw
        bufs = (buf0, buf1, buf2, buf3)
        gsems = (gsem0, gsem1, gsem2, gsem3)
        ssems = (ssem0, ssem1, ssem2, ssem3)
        nb = len(bufs)

        @pl.when(sid == 0)
        def _stage_table():
            pltpu.sync_copy(table_hbm, tab_sh)

        plsc.subcore_barrier()

        def zero_span(lo, cnt):
            def zstep(i, carry):
                idx_v[pl.ds(lo + i * _LANES, _LANES)] = jnp.zeros(
                    (_LANES,), jnp.int32)
                return carry
            lax.fori_loop(0, cnt // _LANES, zstep, 0)

        unroll = 5 * _LANES

        def gath(ci, b):
            idx_chunk = idx_v.at[pl.ds(ci * _CH, _CH)]
            return pltpu.async_copy(tab_sh.at[idx_chunk], bufs[b], gsems[b])

        def fuse_span(lo, cnt):
            def step(i, carry):
                for j in range(unroll // _LANES):
                    s = lo + i * unroll + j * _LANES
                    a = jnp.clip(atv[pl.ds(s, _LANES)], 0, _NT - 1)
                    h = jnp.clip(hyv[pl.ds(s, _LANES)], 0, _NH - 1)
                    f = jnp.clip(fcv[pl.ds(s, _LANES)] + 5, 0, _NC - 1)
                    idx_v[pl.ds(s, _LANES)] = a * (_NH * _NC) + h * _NC + f
                return carry

            lax.fori_loop(0, cnt // unroll, step, 0)

        def prologue(cnt):
            h1 = pltpu.async_copy(
                at_hbm.at[pl.ds(base, cnt)], atv.at[pl.ds(0, cnt)], gsem1)
            h2 = pltpu.async_copy(
                hy_hbm.at[pl.ds(base, cnt)], hyv.at[pl.ds(0, cnt)], gsem1)
            h3 = pltpu.async_copy(
                fc_hbm.at[pl.ds(base, cnt)], fcv.at[pl.ds(0, cnt)], gsem1)
            h1.wait()
            h2.wait()
            h3.wait()
            first = min(cnt, _CH)
            fuse_span(0, first)
            if cnt < _CH:
                zero_span(cnt, _CH - cnt)
            gath(0, 0)
            if cnt > _CH:
                fuse_span(_CH, cnt - _CH)
            if apw - max(cnt, _CH) > 0:
                zero_span(max(cnt, _CH), apw - max(cnt, _CH))

        @pl.when(wid < w0)
        def _full():
            prologue(apw)

        if rem:
            @pl.when(wid == w0)
            def _partial():
                prologue(rem)

        if rem or w0 < _NW:
            @pl.when(wid > w0)
            def _empty():
                zero_span(0, apw)
                gath(0, 0)

        def scat_desc(ci, b):
            return pltpu.make_async_copy(
                bufs[b], out_hbm.at[pl.ds(base + ci * _CH, _CH)], ssems[b])

        def is_real(ci):
            return base + ci * _CH < nout

        def gath_wait(ci, b):
            pltpu.make_async_copy(
                tab_sh.at[idx_v.at[pl.ds(ci * _CH, _CH)]], bufs[b],
                gsems[b]).wait()

        for ci in range(nch):
            b = ci % nb
            prev = ci + 1 - nb
            if prev >= 0:
                @pl.when(is_real(prev))
                def _drain(prev=prev):
                    scat_desc(prev, prev % nb).wait()
            if ci + 1 < nch:
                gath(ci + 1, (ci + 1) % nb)
            gath_wait(ci, b)

            @pl.when(is_real(ci))
            def _emit(ci=ci, b=b):
                scat_desc(ci, b).start()

        for ci in range(max(nch - nb + 1, 0), nch):
            @pl.when(is_real(ci))
            def _drain_tail(ci=ci):
                scat_desc(ci, ci % nb).wait()

    return sc_gather


def kernel(atom_types, hybridization, formal_charges, atom_emb_w, hyb_emb_w,
           charge_emb_w, prop_w, prop_b, out_w, out_b):
    n = atom_types.shape[0]
    group = _NW * _CH
    npad = ((n + group - 1) // group) * group
    props = jnp.stack([
        jnp.asarray(_ATOMIC_NUMBERS, jnp.float32),
        jnp.asarray(_ELECTRONEG, jnp.float32),
        jnp.asarray(_RADII, jnp.float32),
    ], axis=-1)
    table = pl.pallas_call(
        _table_body,
        out_shape=jax.ShapeDtypeStruct((_TPAD, _D), jnp.float32),
    )(atom_emb_w, hyb_emb_w, charge_emb_w, props, prop_w, prop_b, out_w, out_b)
    at = atom_types.astype(jnp.int32)
    hy = hybridization.astype(jnp.int32)
    fc = formal_charges.astype(jnp.int32)
    if n % _CH == 0:
        return _make_sc_gather(npad, n)(table, at, hy, fc)
    atp = jnp.pad(at, (0, npad - n))
    hyp = jnp.pad(hy, (0, npad - n))
    fcp = jnp.pad(fc, (0, npad - n))
    return _make_sc_gather(npad, npad)(table, atp, hyp, fcp)[:n]


# speedup vs baseline: 1.0041x; 1.0041x over previous
import functools

import jax
import jax.numpy as jnp
from jax import lax
from jax.experimental import pallas as pl
from jax.experimental.pallas import tpu as pltpu
from jax.experimental.pallas import tpu_sc as plsc

_ATOMIC_NUMBERS = [0.0, 1.0, 6.0, 7.0, 8.0, 9.0, 15.0, 16.0, 17.0, 35.0, 53.0, 5.0, 14.0, 34.0, 33.0, 26.0, 30.0, 20.0, 12.0, 11.0, 19.0, 25.0, 29.0]
_ELECTRONEG = [2.0, 2.2, 2.55, 3.04, 3.44, 3.98, 2.19, 2.58, 3.16, 2.96, 2.66, 2.04, 1.9, 2.55, 2.18, 1.83, 1.65, 1.0, 1.31, 0.93, 0.82, 1.55, 1.9]
_RADII = [1.7, 1.2, 1.7, 1.55, 1.52, 1.47, 1.8, 1.8, 1.75, 1.85, 1.98, 1.92, 2.1, 1.9, 1.85, 2.0, 1.39, 2.31, 1.73, 2.27, 2.75, 2.05, 1.4]

_D = 128
_D4 = 32
_NT, _NH, _NC = 23, 5, 11
_TROWS = _NT * _NH * _NC
_TPAD = 1280
_NCORE, _NSUB, _LANES = 2, 16, 16
_NW = _NCORE * _NSUB
_CH = 160


def _table_body(aw_ref, hw_ref, cw_ref, props_ref, pw_ref, pb_ref, ow_ref,
                ob_ref, table_ref):
    ow = ow_ref[...]
    w1 = ow[:, 0 * _D4:1 * _D4]
    w2 = ow[:, 1 * _D4:2 * _D4]
    w3 = ow[:, 2 * _D4:3 * _D4]
    w4 = ow[:, 3 * _D4:4 * _D4]
    f32 = jnp.float32
    dot = functools.partial(jnp.dot, preferred_element_type=f32)
    prop_emb = dot(props_ref[...], pw_ref[...].T) + pb_ref[...].reshape(1, _D4)
    a_tab = (dot(aw_ref[...], w1.T) + dot(prop_emb, w2.T)
             + ob_ref[...].reshape(1, _D))
    h_tab = dot(hw_ref[...], w3.T)
    c_tab = dot(cw_ref[...], w4.T)
    a_x = jnp.broadcast_to(a_tab[:, None, :], (_NT, _NH * _NC, _D))
    h_x = jnp.broadcast_to(h_tab[:, None, :], (_NH, _NC, _D)).reshape(
        _NH * _NC, _D)
    c_x = (h_x + jnp.broadcast_to(c_tab[None, :, :], (_NH, _NC, _D)).reshape(
        _NH * _NC, _D))
    full = (a_x + c_x[None, :, :]).reshape(_TROWS, _D)
    table_ref[...] = jnp.concatenate(
        [full, jnp.zeros((_TPAD - _TROWS, _D), f32)], axis=0)


def _make_sc_gather(npad, nout):
    apw = npad // _NW
    nch = apw // _CH
    w0 = nout // apw
    rem = nout % apw
    mesh = plsc.VectorSubcoreMesh(core_axis_name="c", subcore_axis_name="s")

    @functools.partial(
        pl.kernel,
        mesh=mesh,
        out_type=jax.ShapeDtypeStruct((nout, _D), jnp.float32),
        scratch_types=[
            pltpu.VMEM((apw,), jnp.int32),
            pltpu.VMEM((apw,), jnp.int32),
            pltpu.VMEM((apw,), jnp.int32),
            pltpu.VMEM((apw,), jnp.int32),
            pltpu.VMEM((_CH, _D), jnp.float32),
            pltpu.VMEM((_CH, _D), jnp.float32),
            pltpu.VMEM((_CH, _D), jnp.float32),
            pltpu.VMEM_SHARED((_TPAD, _D), jnp.float32),
            pltpu.SemaphoreType.DMA,
            pltpu.SemaphoreType.DMA,
            pltpu.SemaphoreType.DMA,
            pltpu.SemaphoreType.DMA,
            pltpu.SemaphoreType.DMA,
            pltpu.SemaphoreType.DMA,
        ],
    )
    def sc_gather(table_hbm, at_hbm, hy_hbm, fc_hbm, out_hbm,
                  atv, hyv, fcv, idx_v, buf0, buf1, buf2, tab_sh,
                  gsem0, gsem1, gsem2, ssem0, ssem1, ssem2):
        sid = lax.axis_index("s")
        wid = sid * _NCORE + lax.axis_index("c")
        base = wid * apw
        bufs = (buf0, buf1, buf2)
        gsems = (gsem0, gsem1, gsem2)
        ssems = (ssem0, ssem1, ssem2)
        nb = len(bufs)

        @pl.when(sid == 0)
        def _stage_table():
            pltpu.sync_copy(table_hbm, tab_sh)

        plsc.subcore_barrier()

        def zero_span(lo, cnt):
            def zstep(i, carry):
                idx_v[pl.ds(lo + i * _LANES, _LANES)] = jnp.zeros(
                    (_LANES,), jnp.int32)
                return carry
            lax.fori_loop(0, cnt // _LANES, zstep, 0)

        unroll = 5 * _LANES

        def gath(ci, b):
            idx_chunk = idx_v.at[pl.ds(ci * _CH, _CH)]
            return pltpu.async_copy(tab_sh.at[idx_chunk], bufs[b], gsems[b])

        def fuse_span(lo, cnt):
            def step(i, carry):
                for j in range(unroll // _LANES):
                    s = lo + i * unroll + j * _LANES
                    a = jnp.clip(atv[pl.ds(s, _LANES)], 0, _NT - 1)
                    h = jnp.clip(hyv[pl.ds(s, _LANES)], 0, _NH - 1)
                    f = jnp.clip(fcv[pl.ds(s, _LANES)] + 5, 0, _NC - 1)
                    idx_v[pl.ds(s, _LANES)] = a * (_NH * _NC) + h * _NC + f
                return carry

            lax.fori_loop(0, cnt // unroll, step, 0)

        def prologue(cnt):
            h1 = pltpu.async_copy(
                at_hbm.at[pl.ds(base, cnt)], atv.at[pl.ds(0, cnt)], gsem1)
            h2 = pltpu.async_copy(
                hy_hbm.at[pl.ds(base, cnt)], hyv.at[pl.ds(0, cnt)], gsem1)
            h3 = pltpu.async_copy(
                fc_hbm.at[pl.ds(base, cnt)], fcv.at[pl.ds(0, cnt)], gsem1)
            h1.wait()
            h2.wait()
            h3.wait()
            first = min(cnt, _CH)
            fuse_span(0, first)
            if cnt < _CH:
                zero_span(cnt, _CH - cnt)
            gath(0, 0)
            if cnt > _CH:
                fuse_span(_CH, cnt - _CH)
            if apw - max(cnt, _CH) > 0:
                zero_span(max(cnt, _CH), apw - max(cnt, _CH))

        @pl.when(wid < w0)
        def _full():
            prologue(apw)

        if rem:
            @pl.when(wid == w0)
            def _partial():
                prologue(rem)

        if rem or w0 < _NW:
            @pl.when(wid > w0)
            def _empty():
                zero_span(0, apw)
                gath(0, 0)

        def scat_desc(ci, b):
            return pltpu.make_async_copy(
                bufs[b], out_hbm.at[pl.ds(base + ci * _CH, _CH)], ssems[b])

        def is_real(ci):
            return base + ci * _CH < nout

        def gath_wait(ci, b):
            pltpu.make_async_copy(
                tab_sh.at[idx_v.at[pl.ds(ci * _CH, _CH)]], bufs[b],
                gsems[b]).wait()

        for ci in range(nch):
            b = ci % nb
            prev = ci + 1 - nb
            if prev >= 0:
                @pl.when(is_real(prev))
                def _drain(prev=prev):
                    scat_desc(prev, prev % nb).wait()
            if ci + 1 < nch:
                gath(ci + 1, (ci + 1) % nb)
            gath_wait(ci, b)

            @pl.when(is_real(ci))
            def _emit(ci=ci, b=b):
                scat_desc(ci, b).start()

        for ci in range(max(nch - nb + 1, 0), nch):
            @pl.when(is_real(ci))
            def _drain_tail(ci=ci):
                scat_desc(ci, ci % nb).wait()

    return sc_gather


def kernel(atom_types, hybridization, formal_charges, atom_emb_w, hyb_emb_w,
           charge_emb_w, prop_w, prop_b, out_w, out_b):
    n = atom_types.shape[0]
    group = _NW * _CH
    npad = ((n + group - 1) // group) * group
    props = jnp.stack([
        jnp.asarray(_ATOMIC_NUMBERS, jnp.float32),
        jnp.asarray(_ELECTRONEG, jnp.float32),
        jnp.asarray(_RADII, jnp.float32),
    ], axis=-1)
    table = pl.pallas_call(
        _table_body,
        out_shape=jax.ShapeDtypeStruct((_TPAD, _D), jnp.float32),
    )(atom_emb_w, hyb_emb_w, charge_emb_w, props, prop_w, prop_b, out_w, out_b)
    at = atom_types.astype(jnp.int32)
    hy = hybridization.astype(jnp.int32)
    fc = formal_charges.astype(jnp.int32)
    if n % _CH == 0:
        return _make_sc_gather(npad, n)(table, at, hy, fc)
    atp = jnp.pad(at, (0, npad - n))
    hyp = jnp.pad(hy, (0, npad - n))
    fcp = jnp.pad(fc, (0, npad - n))
    return _make_sc_gather(npad, npad)(table, atp, hyp, fcp)[:n]
